# baseline (speedup 1.0000x reference)
import functools

import jax
import jax.numpy as jnp
from jax import lax
from jax.experimental import pallas as pl
from jax.experimental.pallas import tpu as pltpu
from jax.experimental.pallas import tpu_sc as plsc

BATCH = 16384
EMB = 16
NC = 2
NS = 16
NW = NC * NS
BPW = BATCH // NW
L = 16
CH = 128
NCHUNK = BPW // CH


def _mesh():
    return plsc.VectorSubcoreMesh(core_axis_name="c", subcore_axis_name="s")


@functools.partial(
    pl.kernel,
    out_type=jax.ShapeDtypeStruct((NW * EMB,), jnp.float32),
    mesh=_mesh(),
    compiler_params=pltpu.CompilerParams(needs_layout_passes=False),
    scratch_types=[
        pltpu.VMEM((BPW,), jnp.int32),
        pltpu.VMEM((BPW,), jnp.int32),
        pltpu.VMEM((BPW,), jnp.int32),
        pltpu.VMEM((BPW,), jnp.int32),
        pltpu.VMEM((CH, 128), jnp.float32),
        pltpu.VMEM((CH, 128), jnp.float32),
        pltpu.VMEM((CH, 128), jnp.float32),
        pltpu.VMEM((CH, 128), jnp.float32),
        pltpu.VMEM((EMB,), jnp.float32),
        pltpu.SemaphoreType.DMA,
        pltpu.SemaphoreType.DMA,
        pltpu.SemaphoreType.DMA,
        pltpu.SemaphoreType.DMA,
    ],
)
def _gather_partials(uidx_hbm, midx_hbm, ue_hbm, me_hbm, part_hbm,
                     urow_v, mrow_v, uoff_v, moff_v,
                     ubuf0, ubuf1, mbuf0, mbuf1, acc_v,
                     semu0, semu1, semm0, semm1):
    wid = lax.axis_index("s") * NC + lax.axis_index("c")
    base = wid * BPW
    pltpu.sync_copy(uidx_hbm.at[pl.ds(base, BPW)], urow_v)
    pltpu.sync_copy(midx_hbm.at[pl.ds(base, BPW)], mrow_v)
    for i in range(BPW // L):
        sl = pl.ds(i * L, L)
        u = urow_v[sl]
        m = mrow_v[sl]
        uoff_v[sl] = lax.shift_left(jnp.bitwise_and(u, 7), 4)
        moff_v[sl] = lax.shift_left(jnp.bitwise_and(m, 7), 4)
        urow_v[sl] = lax.shift_right_logical(u, 3)
        mrow_v[sl] = lax.shift_right_logical(m, 3)

    ubufs = (ubuf0, ubuf1)
    mbufs = (mbuf0, mbuf1)
    usems = (semu0, semu1)
    msems = (semm0, semm1)

    def fire(c):
        sl = pl.ds(c * CH, CH)
        cu = pltpu.async_copy(ue_hbm.at[urow_v.at[sl]], ubufs[c % 2], usems[c % 2])
        cm = pltpu.async_copy(me_hbm.at[mrow_v.at[sl]], mbufs[c % 2], msems[c % 2])
        return cu, cm

    inflight = fire(0)
    acc = jnp.zeros((L,), jnp.float32)
    rowi = lax.iota(jnp.int32, L)
    for c in range(NCHUNK):
        cu, cm = inflight
        if c + 1 < NCHUNK:
            nxt = fire(c + 1)
        cu.wait()
        cm.wait()
        ub = ubufs[c % 2]
        mb = mbufs[c % 2]

        def gbody(g, acc, _c=c, _ub=ub, _mb=mb):
            ri = rowi + g * L
            uo = plsc.load_gather(uoff_v, [ri + _c * CH])
            mo = plsc.load_gather(moff_v, [ri + _c * CH])
            for j in range(L):
                uj = plsc.load_gather(_ub, [ri, uo + j])
                mj = plsc.load_gather(_mb, [ri, mo + j])
                acc = acc + uj * mj
            return acc

        acc = lax.fori_loop(0, CH // L, gbody, acc)
        if c + 1 < NCHUNK:
            inflight = nxt
    acc_v[...] = acc
    pltpu.sync_copy(acc_v, part_hbm.at[pl.ds(wid * EMB, EMB)])


@functools.partial(
    pl.kernel,
    out_type=jax.ShapeDtypeStruct((BATCH,), jnp.float32),
    mesh=_mesh(),
    compiler_params=pltpu.CompilerParams(use_tc_tiling_on_sc=False,
                                         needs_layout_passes=False),
    scratch_types=[
        pltpu.VMEM((NW * EMB,), jnp.float32),
        pltpu.VMEM((BPW,), jnp.int32),
        pltpu.VMEM((BPW,), jnp.int32),
        pltpu.VMEM((BPW,), jnp.float32),
        pltpu.VMEM((BPW,), jnp.float32),
        pltpu.VMEM((BPW,), jnp.float32),
        pltpu.SemaphoreType.DMA,
        pltpu.SemaphoreType.DMA,
        pltpu.SemaphoreType.DMA,
    ],
)
def _reduce_sigmoid(part_hbm, uidx_hbm, midx_hbm, ub_hbm, mb_hbm, out_hbm,
                    part_v, uidx_v, midx_v, ub_v, mb_v, out_v,
                    sem0, sem1, sem2):
    wid = lax.axis_index("s") * NC + lax.axis_index("c")
    base = wid * BPW
    cp = pltpu.async_copy(part_hbm, part_v, sem2)
    pltpu.sync_copy(uidx_hbm.at[pl.ds(base, BPW)], uidx_v)
    pltpu.sync_copy(midx_hbm.at[pl.ds(base, BPW)], midx_v)
    c0 = pltpu.async_copy(ub_hbm.at[uidx_v], ub_v, sem0)
    c1 = pltpu.async_copy(mb_hbm.at[midx_v], mb_v, sem1)
    cp.wait()
    acc = part_v[pl.ds(0, L)]
    for j in range(1, NW):
        acc = acc + part_v[pl.ds(j * EMB, L)]
    total = lax.reduce_sum_p.bind(acc, axes=(0,))
    c0.wait()
    c1.wait()
    for i in range(BPW // L):
        sl = pl.ds(i * L, L)
        x = ub_v[sl] + mb_v[sl] + total
        out_v[sl] = 1.0 / (1.0 + jnp.exp(-x))
    pltpu.sync_copy(out_v, out_hbm.at[pl.ds(base, BPW)])


def kernel(inputs, user_embedding, user_bias, movie_embedding, movie_bias):
    uidx = inputs[:, 0]
    midx = inputs[:, 1]
    ue128 = user_embedding.reshape(-1, 128)
    me128 = movie_embedding.reshape(-1, 128)
    part = _gather_partials(uidx, midx, ue128, me128)
    out = _reduce_sigmoid(part, uidx, midx, user_bias.reshape(-1),
                          movie_bias.reshape(-1))
    return out.reshape(BATCH, 1)
